# Initial kernel scaffold; baseline (speedup 1.0000x reference)
import jax
import jax.numpy as jnp
from jax.experimental import pallas as pl


def kernel(x, gate_w, gate_b, w1, b1, w2, b2):
    raise NotImplementedError("write your pallas kernel here")


# speedup vs baseline: 3.8703x; 3.8703x over previous
import jax
import jax.numpy as jnp
from jax.experimental import pallas as pl
from jax.experimental.pallas import tpu as pltpu

_B, _T, _D, _E, _DF = 64, 256, 384, 4, 1536
_N = _B * _T
_TM = 256
_EP = 128


def _moe_kernel(x_ref, gw_ref, gb_ref, w1_ref, b1_ref, w2_ref, b2_ref,
                out_ref, bal_ref):
    xt = x_ref[...]
    scores = jnp.dot(xt, gw_ref[...], preferred_element_type=jnp.float32)
    scores = scores + gb_ref[...]
    scores = jnp.nan_to_num(scores, nan=0.0)
    m = jnp.max(scores, axis=1, keepdims=True)
    ex = jnp.exp(scores - m)
    probs = ex / jnp.sum(ex, axis=1, keepdims=True)

    psum = jnp.sum(probs, axis=0, keepdims=True)
    @pl.when(pl.program_id(0) == 0)
    def _init():
        bal_ref[...] = jnp.zeros_like(bal_ref)
    bal_ref[...] += jnp.broadcast_to(psum, bal_ref.shape)

    lane = jax.lax.broadcasted_iota(jnp.int32, probs.shape, 1)
    m1 = jnp.max(probs, axis=1, keepdims=True)
    a1 = jnp.min(jnp.where(probs == m1, lane, _EP), axis=1, keepdims=True)
    p2 = jnp.where(lane == a1, -jnp.inf, probs)
    m2 = jnp.max(p2, axis=1, keepdims=True)
    a2 = jnp.min(jnp.where(p2 == m2, lane, _EP), axis=1, keepdims=True)
    sel = (lane == a1) | (lane == a2)
    masked = jnp.where(sel, probs, 0.0)
    wgt = masked / (jnp.sum(masked, axis=1, keepdims=True) + 1e-9)

    acc = jnp.zeros((_TM, _D), dtype=jnp.float32)
    for e in range(_E):
        h = jnp.dot(xt, w1_ref[e], preferred_element_type=jnp.float32)
        h = h + b1_ref[e]
        h = 0.5 * h * (1.0 + jax.lax.erf(h * 0.7071067811865476))
        y = jnp.dot(h, w2_ref[e], preferred_element_type=jnp.float32)
        acc = acc + wgt[:, e:e + 1] * (y + b2_ref[e])
    out_ref[...] = acc


def kernel(x, gate_w, gate_b, w1, b1, w2, b2):
    x2 = x.reshape(_N, _D)
    gw_p = jnp.zeros((_D, _EP), jnp.float32).at[:, :_E].set(gate_w)
    gb_p = jnp.full((1, _EP), -jnp.inf, jnp.float32).at[0, :_E].set(gate_b)

    grid = _N // _TM
    out, bal = pl.pallas_call(
        _moe_kernel,
        grid=(grid,),
        in_specs=[
            pl.BlockSpec((_TM, _D), lambda i: (i, 0)),
            pl.BlockSpec((_D, _EP), lambda i: (0, 0)),
            pl.BlockSpec((1, _EP), lambda i: (0, 0)),
            pl.BlockSpec((_E, _D, _DF), lambda i: (0, 0, 0)),
            pl.BlockSpec((_E, 1, _DF), lambda i: (0, 0, 0)),
            pl.BlockSpec((_E, _DF, _D), lambda i: (0, 0, 0)),
            pl.BlockSpec((_E, 1, _D), lambda i: (0, 0, 0)),
        ],
        out_specs=[
            pl.BlockSpec((_TM, _D), lambda i: (i, 0)),
            pl.BlockSpec((8, _EP), lambda i: (0, 0)),
        ],
        out_shape=[
            jax.ShapeDtypeStruct((_N, _D), jnp.float32),
            jax.ShapeDtypeStruct((8, _EP), jnp.float32),
        ],
    )(x2, gw_p, gb_p, w1, b1.reshape(_E, 1, _DF), w2, b2.reshape(_E, 1, _D))

    bl = (jnp.sum((bal[0, :_E] / _N) ** 2)) * _E
    bal_loss = jnp.clip(bl, 0.0, 5.0)
    return out.reshape(_B, _T, _D), bal_loss
